# baseline (speedup 1.0000x reference)
import functools

import numpy as np
import jax
import jax.numpy as jnp
from jax import lax
from jax.experimental import pallas as pl
from jax.experimental.pallas import tpu as pltpu
from jax.experimental.pallas import tpu_sc as plsc

N_V = 100000
NP = 100352
N_F = 200000
E = 6 * N_F
NTILES = 32
CHUNK = 1024
NCHUNK = 37
EPT = NCHUNK * CHUNK
EPAD = EPT * NTILES
CPT = NP // 16
RW = 8


def _sc_scatter(vert_pad, zeros, src_idx, dst_idx):
    mesh = plsc.VectorSubcoreMesh(core_axis_name="c", subcore_axis_name="s")

    @functools.partial(
        pl.kernel,
        mesh=mesh,
        compiler_params=pltpu.CompilerParams(use_tc_tiling_on_sc=False),
        out_type=jax.ShapeDtypeStruct((2, NP, RW), jnp.float32),
        scratch_types=[
            pltpu.VMEM_SHARED((NP, RW), jnp.float32),
            pltpu.VMEM((CHUNK,), jnp.int32),
            pltpu.VMEM((CHUNK,), jnp.int32),
            pltpu.VMEM((CHUNK, RW), jnp.float32),
        ],
    )
    def body(vp_hbm, z_hbm, src_hbm, dst_hbm, out_hbm,
             acc_sh, srcv, dstv, rows):
        cid = lax.axis_index("c")
        sid = lax.axis_index("s")
        wid = sid * 2 + cid
        r0 = sid * CPT

        pltpu.sync_copy(z_hbm.at[pl.ds(r0, CPT)], acc_sh.at[pl.ds(r0, CPT)])
        plsc.subcore_barrier()

        def outer(j, carry):
            pltpu.sync_copy(src_hbm.at[wid, j], srcv)
            pltpu.sync_copy(dst_hbm.at[wid, j], dstv)
            pltpu.sync_copy(vp_hbm.at[srcv], rows)
            pltpu.sync_copy(rows, acc_sh.at[dstv], add=True)
            return carry

        lax.fori_loop(0, NCHUNK, outer, 0)
        plsc.subcore_barrier()

        pltpu.sync_copy(acc_sh.at[pl.ds(r0, CPT)],
                        out_hbm.at[cid, pl.ds(r0, CPT)])

    return body(vert_pad, zeros, src_idx, dst_idx)


def _finalize_body(p, v, tdeg, tsum, o):
    x = p[0] + p[1]
    deg = jnp.maximum(
        jnp.dot(x, tdeg[...], preferred_element_type=jnp.float32,
                precision=lax.Precision.HIGHEST), 1.0)
    lap = x / deg - v[...]
    sq = lap * lap
    o[...] = jnp.sqrt(
        jnp.dot(sq, tsum[...], preferred_element_type=jnp.float32,
                precision=lax.Precision.HIGHEST))


def kernel(vert, faces):
    i0 = faces[:, 0]
    i1 = faces[:, 1]
    i2 = faces[:, 2]
    src = jnp.concatenate([i0, i1, i1, i2, i2, i0])
    dst = jnp.concatenate([i1, i0, i2, i1, i0, i2])
    src = jnp.pad(src, (0, EPAD - E), constant_values=N_V)
    dst = jnp.pad(dst, (0, EPAD - E), constant_values=N_V)
    src = src.reshape(NTILES, NCHUNK, CHUNK)
    dst = dst.reshape(NTILES, NCHUNK, CHUNK)

    vert_pad = jnp.concatenate(
        [vert, jnp.ones((N_V, 1), jnp.float32),
         jnp.zeros((N_V, RW - 4), jnp.float32)], axis=1)
    vert_pad = jnp.pad(vert_pad, ((0, NP - N_V), (0, 0)))
    zeros = jnp.zeros((NP, RW), jnp.float32)

    part = _sc_scatter(vert_pad, zeros, src, dst)

    nr = NP * RW // 128
    lanes = np.arange(128)
    tdeg = jnp.asarray(
        (lanes[:, None] == 8 * (lanes[None, :] // 8) + 3).astype(np.float32))
    tsum = jnp.asarray(
        ((lanes[:, None] // 8 == np.arange(16)[None, :])
         & (lanes[:, None] % 8 < 3)).astype(np.float32))
    curve = pl.pallas_call(
        _finalize_body,
        out_shape=jax.ShapeDtypeStruct((nr, 16), jnp.float32),
    )(part.reshape(2, nr, 128), vert_pad.reshape(nr, 128), tdeg, tsum)
    return curve.reshape(NP)[:N_V]


# speedup vs baseline: 1.6532x; 1.6532x over previous
import functools

import numpy as np
import jax
import jax.numpy as jnp
from jax import lax
from jax.experimental import pallas as pl
from jax.experimental.pallas import tpu as pltpu
from jax.experimental.pallas import tpu_sc as plsc

N_V = 100000
NP = 100352
N_F = 200000
NTILES = 32
FPT = 6272
NF_PAD = FPT * NTILES
CHUNK = 1024
NFULL = 6
REM = FPT - NFULL * CHUNK
CPT = NP // 16
RW = 8

_COLS = ((0, (1, 2)), (1, (0, 2)), (2, (1, 0)))


def _sc_scatter(vert_pad, zeros, faces_t):
    mesh = plsc.VectorSubcoreMesh(core_axis_name="c", subcore_axis_name="s")

    @functools.partial(
        pl.kernel,
        mesh=mesh,
        compiler_params=pltpu.CompilerParams(use_tc_tiling_on_sc=False),
        out_type=jax.ShapeDtypeStruct((2, NP, RW), jnp.float32),
        scratch_types=[
            pltpu.VMEM_SHARED((NP, RW), jnp.float32),
            pltpu.VMEM((CHUNK,), jnp.int32),
            pltpu.VMEM((CHUNK,), jnp.int32),
            pltpu.VMEM((CHUNK,), jnp.int32),
            pltpu.VMEM((CHUNK, RW), jnp.float32),
            pltpu.VMEM((REM,), jnp.int32),
            pltpu.VMEM((REM,), jnp.int32),
            pltpu.VMEM((REM,), jnp.int32),
            pltpu.VMEM((REM, RW), jnp.float32),
        ],
    )
    def body(vp_hbm, z_hbm, ft_hbm, out_hbm,
             acc_sh, srcv, dstv, dstv2, rows, srcr, dstr, dstr2, rowsr):
        cid = lax.axis_index("c")
        sid = lax.axis_index("s")
        wid = sid * 2 + cid
        r0 = sid * CPT
        fbase = wid * FPT

        pltpu.sync_copy(z_hbm.at[pl.ds(r0, CPT)], acc_sh.at[pl.ds(r0, CPT)])
        plsc.subcore_barrier()

        for cs, (cd0, cd1) in _COLS:
            def inner(c, carry):
                f0 = fbase + c * CHUNK
                pltpu.sync_copy(ft_hbm.at[cs, pl.ds(f0, CHUNK)], srcv)
                pltpu.sync_copy(ft_hbm.at[cd0, pl.ds(f0, CHUNK)], dstv)
                pltpu.sync_copy(ft_hbm.at[cd1, pl.ds(f0, CHUNK)], dstv2)
                pltpu.sync_copy(vp_hbm.at[srcv], rows)
                pltpu.sync_copy(rows, acc_sh.at[dstv], add=True)
                pltpu.sync_copy(rows, acc_sh.at[dstv2], add=True)
                return carry

            lax.fori_loop(0, NFULL, inner, 0)
            f0 = fbase + NFULL * CHUNK
            pltpu.sync_copy(ft_hbm.at[cs, pl.ds(f0, REM)], srcr)
            pltpu.sync_copy(ft_hbm.at[cd0, pl.ds(f0, REM)], dstr)
            pltpu.sync_copy(ft_hbm.at[cd1, pl.ds(f0, REM)], dstr2)
            pltpu.sync_copy(vp_hbm.at[srcr], rowsr)
            pltpu.sync_copy(rowsr, acc_sh.at[dstr], add=True)
            pltpu.sync_copy(rowsr, acc_sh.at[dstr2], add=True)

        plsc.subcore_barrier()
        pltpu.sync_copy(acc_sh.at[pl.ds(r0, CPT)],
                        out_hbm.at[cid, pl.ds(r0, CPT)])

    return body(vert_pad, zeros, faces_t)


def _finalize_body(p, v, tdeg, tsum, o):
    x = p[0] + p[1]
    deg = jnp.maximum(
        jnp.dot(x, tdeg[...], preferred_element_type=jnp.float32,
                precision=lax.Precision.HIGHEST), 1.0)
    lap = x / deg - v[...]
    sq = lap * lap
    o[...] = jnp.sqrt(
        jnp.dot(sq, tsum[...], preferred_element_type=jnp.float32,
                precision=lax.Precision.HIGHEST))


def kernel(vert, faces):
    faces_t = jnp.pad(faces, ((0, NF_PAD - N_F), (0, 0)),
                      constant_values=N_V).T

    vert_pad = jnp.concatenate(
        [vert, jnp.ones((N_V, 1), jnp.float32),
         jnp.zeros((N_V, RW - 4), jnp.float32)], axis=1)
    vert_pad = jnp.pad(vert_pad, ((0, NP - N_V), (0, 0)))
    zeros = jnp.zeros((NP, RW), jnp.float32)

    part = _sc_scatter(vert_pad, zeros, faces_t)

    nr = NP * RW // 128
    lanes = np.arange(128)
    tdeg = jnp.asarray(
        (lanes[:, None] == 8 * (lanes[None, :] // 8) + 3).astype(np.float32))
    tsum = jnp.asarray(
        ((lanes[:, None] // 8 == np.arange(16)[None, :])
         & (lanes[:, None] % 8 < 3)).astype(np.float32))
    curve = pl.pallas_call(
        _finalize_body,
        out_shape=jax.ShapeDtypeStruct((nr, 16), jnp.float32),
    )(part.reshape(2, nr, 128), vert_pad.reshape(nr, 128), tdeg, tsum)
    return curve.reshape(NP)[:N_V]
